# Initial kernel scaffold; baseline (speedup 1.0000x reference)
import jax
import jax.numpy as jnp
from jax.experimental import pallas as pl


def kernel(x, W1, W2, Wd1, bd1, Wd2, bd2, edge_index, edge_type):
    raise NotImplementedError("write your pallas kernel here")


# speedup vs baseline: 7.7546x; 7.7546x over previous
import functools

import jax
import jax.numpy as jnp
from jax import lax
from jax.experimental import pallas as pl
from jax.experimental.pallas import tpu as pltpu
from jax.experimental.pallas import tpu_sc as plsc

_N = 10000
_E = 320000
_NFEAT = 128
_NEMB = 126
_PER = 42
_R = 3
_DW = 48
_RN = _R * _N
_RNP = 30720
_NW = 32
_EW = _E // _NW
_CH = 80
_NCHUNK = _EW // _CH
_ZROWS = _RNP // 16
_ZCH = 120
_NZ = _ZROWS // _ZCH

_mesh = plsc.VectorSubcoreMesh(core_axis_name="c", subcore_axis_name="s")


@functools.partial(
    pl.kernel,
    mesh=_mesh,
    out_type=jax.ShapeDtypeStruct((2 * _RNP, _DW), jnp.float32),
    scratch_types=[
        pltpu.VMEM((_CH,), jnp.int32),
        pltpu.VMEM((_CH,), jnp.int32),
        pltpu.VMEM((_CH, _DW), jnp.float32),
        pltpu.VMEM((_ZCH, _DW), jnp.float32),
        pltpu.VMEM_SHARED((_RNP, _DW), jnp.float32),
        pltpu.SemaphoreType.DMA,
    ],
    compiler_params=pltpu.CompilerParams(use_tc_tiling_on_sc=False),
)
def _edge_pass(table, isrc, idst, out, isrc_v, idst_v, rows_v, zbuf, acc, sem):
    cid = lax.axis_index("c")
    sid = lax.axis_index("s")
    wid = cid * 16 + sid
    zero16 = jnp.zeros((16,), jnp.float32)

    def _zrow(r, carry):
        for cpart in range(_DW // 16):
            zbuf[r, pl.ds(cpart * 16, 16)] = zero16
        return carry

    lax.fori_loop(0, _ZCH, _zrow, 0)

    def _zacc(j, carry):
        pltpu.sync_copy(zbuf, acc.at[pl.ds(sid * _ZROWS + j * _ZCH, _ZCH)])
        return carry

    lax.fori_loop(0, _NZ, _zacc, 0)

    plsc.subcore_barrier()

    def _chunk(g, carry):
        base = wid * _EW + g * _CH
        pltpu.sync_copy(isrc.at[pl.ds(base, _CH)], isrc_v)
        pltpu.sync_copy(idst.at[pl.ds(base, _CH)], idst_v)
        pltpu.async_copy(table.at[isrc_v], rows_v, sem).wait()
        pltpu.sync_copy(rows_v, acc.at[idst_v], add=True)
        return carry

    lax.fori_loop(0, _NCHUNK, _chunk, 0)

    plsc.subcore_barrier()

    def _copy_out(j, carry):
        start = sid * _ZROWS + j * _ZCH
        pltpu.sync_copy(acc.at[pl.ds(start, _ZCH)],
                        out.at[pl.ds(cid * _RNP + start, _ZCH)])
        return carry

    lax.fori_loop(0, _NZ, _copy_out, 0)


def _idx_body(src_ref, dst_ref, et_ref, isrc_ref, idst_ref):
    et = et_ref[...]
    isrc_ref[...] = src_ref[...] * 3 + et
    idst_ref[...] = dst_ref[...] * 3 + et


def _proj_body(x_ref, w_ref, o_ref):
    o_ref[...] = jnp.dot(x_ref[...], w_ref[...],
                         preferred_element_type=jnp.float32,
                         precision=lax.Precision.HIGHEST)


def _mid_body(h0_ref, h1_ref, w_ref, o_ref):
    h = h0_ref[...] + h1_ref[...]
    h = jnp.where(h >= 0.0, h, 0.01 * h)
    o_ref[...] = jnp.dot(h, w_ref[...],
                         preferred_element_type=jnp.float32,
                         precision=lax.Precision.HIGHEST)


def _dec_body(e0_ref, e1_ref, wd1_ref, b1_ref, wd2_ref, b2_ref, x_ref, o_ref):
    emb = e0_ref[...] + e1_ref[...]
    hid = jnp.maximum(
        jnp.dot(emb, wd1_ref[...], preferred_element_type=jnp.float32,
                precision=lax.Precision.HIGHEST) + b1_ref[...], 0.0)
    xh = jnp.dot(hid, wd2_ref[...], preferred_element_type=jnp.float32,
                 precision=lax.Precision.HIGHEST) + b2_ref[...]
    d = xh - x_ref[...]
    part = jnp.sum(d * d)

    @pl.when(pl.program_id(0) == 0)
    def _init():
        o_ref[...] = jnp.zeros_like(o_ref)

    o_ref[...] += jnp.full((1, 1), 1.0, jnp.float32) * part


def _rows(bm, cols):
    return pl.BlockSpec((bm, cols), lambda i: (i, 0))


def _full(r, c):
    return pl.BlockSpec((r, c), lambda i: (0, 0))


def kernel(x, W1, W2, Wd1, bd1, Wd2, bd2, edge_index, edge_type):
    f32 = jnp.float32
    W1p = jnp.pad(W1, ((0, 0), (0, 0), (0, _DW - _PER)))
    W1cat = jnp.transpose(W1p, (1, 0, 2)).reshape(_NFEAT, _R * _DW)
    W2p = jnp.pad(W2, ((0, 0), (0, _DW - _PER), (0, _DW - _PER)))
    W2bd = jax.scipy.linalg.block_diag(W2p[0], W2p[1], W2p[2])
    Wd1p = jnp.pad(Wd1.reshape(_R, _PER, _NEMB),
                   ((0, 0), (0, _DW - _PER), (0, 0))).reshape(_R * _DW, _NEMB)
    Wd1p = jnp.pad(Wd1p, ((0, 0), (0, _NFEAT - _NEMB)))
    b1p = jnp.pad(bd1, (0, _NFEAT - _NEMB)).reshape(1, _NFEAT)
    Wd2p = jnp.pad(Wd2, ((0, _NFEAT - _NEMB), (0, 0)))
    b2p = bd2.reshape(1, _NFEAT)

    rows2d = _E // 128
    src2 = edge_index[0].reshape(rows2d, 128)
    dst2 = edge_index[1].reshape(rows2d, 128)
    et2 = edge_type.reshape(rows2d, 128)

    isrc2, idst2 = pl.pallas_call(
        _idx_body,
        grid=(1,),
        in_specs=[_full(rows2d, 128)] * 3,
        out_specs=[_full(rows2d, 128)] * 2,
        out_shape=[jax.ShapeDtypeStruct((rows2d, 128), jnp.int32)] * 2,
    )(src2, dst2, et2)
    isrc = isrc2.reshape(_E)
    idst = idst2.reshape(_E)

    bm = 2000
    P = pl.pallas_call(
        _proj_body,
        grid=(_N // bm,),
        in_specs=[_rows(bm, _NFEAT), _full(_NFEAT, _R * _DW)],
        out_specs=_rows(bm, _R * _DW),
        out_shape=jax.ShapeDtypeStruct((_N, _R * _DW), f32),
    )(x, W1cat)

    Pt = jnp.pad(P.reshape(_RN, _DW), ((0, _RNP - _RN), (0, 0)))
    Hflat = _edge_pass(Pt, isrc, idst)
    H0 = Hflat[:_RN].reshape(_N, _R * _DW)
    H1 = Hflat[_RNP:_RNP + _RN].reshape(_N, _R * _DW)

    Q = pl.pallas_call(
        _mid_body,
        grid=(_N // bm,),
        in_specs=[_rows(bm, _R * _DW), _rows(bm, _R * _DW),
                  _full(_R * _DW, _R * _DW)],
        out_specs=_rows(bm, _R * _DW),
        out_shape=jax.ShapeDtypeStruct((_N, _R * _DW), f32),
    )(H0, H1, W2bd)

    Qt = jnp.pad(Q.reshape(_RN, _DW), ((0, _RNP - _RN), (0, 0)))
    Eflat = _edge_pass(Qt, isrc, idst)
    E0 = Eflat[:_RN].reshape(_N, _R * _DW)
    E1 = Eflat[_RNP:_RNP + _RN].reshape(_N, _R * _DW)

    ssq = pl.pallas_call(
        _dec_body,
        grid=(_N // bm,),
        in_specs=[_rows(bm, _R * _DW), _rows(bm, _R * _DW),
                  _full(_R * _DW, _NFEAT), _full(1, _NFEAT),
                  _full(_NFEAT, _NFEAT), _full(1, _NFEAT),
                  _rows(bm, _NFEAT)],
        out_specs=_full(1, 1),
        out_shape=jax.ShapeDtypeStruct((1, 1), f32),
    )(E0, E1, Wd1p, b1p, Wd2p, b2p, x)

    sem_loss = ssq[0, 0] / (_N * _NFEAT)
    zero = jnp.asarray(0.0, dtype=f32)
    return jnp.stack([sem_loss, zero, zero, zero])


# speedup vs baseline: 12.5179x; 1.6143x over previous
import functools

import jax
import jax.numpy as jnp
from jax import lax
from jax.experimental import pallas as pl
from jax.experimental.pallas import tpu as pltpu
from jax.experimental.pallas import tpu_sc as plsc

_N = 10000
_E = 320000
_NFEAT = 128
_NEMB = 126
_PER = 42
_R = 3
_DW = 48
_RN = _R * _N
_NP = 10240
_RNP = 30720
_NW = 32
_EW = _E // _NW
_CH = 80
_NCHUNK = _EW // _CH
_ZROWS = _RNP // 16
_ZCH = 120
_NZ = _ZROWS // _ZCH

_mesh = plsc.VectorSubcoreMesh(core_axis_name="c", subcore_axis_name="s")


@functools.partial(
    pl.kernel,
    mesh=_mesh,
    out_type=jax.ShapeDtypeStruct((2 * _RNP, _DW), jnp.float32),
    scratch_types=[
        pltpu.VMEM((_NCHUNK, _CH), jnp.int32),
        pltpu.VMEM((_NCHUNK, _CH), jnp.int32),
        pltpu.VMEM((2, _CH, _DW), jnp.float32),
        pltpu.VMEM((_ZCH, _DW), jnp.float32),
        pltpu.VMEM_SHARED((_RNP, _DW), jnp.float32),
        pltpu.SemaphoreType.DMA((2,)),
    ],
    compiler_params=pltpu.CompilerParams(use_tc_tiling_on_sc=False),
)
def _edge_pass(table, isrc, idst, out, isrc_v, idst_v, rows_v, zbuf, acc, sems):
    cid = lax.axis_index("c")
    sid = lax.axis_index("s")
    wid = cid * 16 + sid
    zero16 = jnp.zeros((16,), jnp.float32)

    pltpu.sync_copy(isrc.at[wid], isrc_v)
    pltpu.sync_copy(idst.at[wid], idst_v)

    def _gather(g, b):
        return pltpu.async_copy(table.at[isrc_v.at[g]], rows_v.at[b],
                                sems.at[b])

    _gather(0, 0)

    def _zrow(r, carry):
        for cpart in range(_DW // 16):
            zbuf[r, pl.ds(cpart * 16, 16)] = zero16
        return carry

    lax.fori_loop(0, _ZCH, _zrow, 0)

    def _zacc(j, carry):
        pltpu.sync_copy(zbuf, acc.at[pl.ds(sid * _ZROWS + j * _ZCH, _ZCH)])
        return carry

    lax.fori_loop(0, _NZ, _zacc, 0)

    plsc.subcore_barrier()

    def _chunk(g, carry):
        b = lax.rem(g, 2)
        pltpu.make_async_copy(table.at[isrc_v.at[g]], rows_v.at[b],
                              sems.at[b]).wait()

        @pl.when(g + 1 < _NCHUNK)
        def _next():
            _gather(g + 1, lax.rem(g + 1, 2))

        pltpu.sync_copy(rows_v.at[b], acc.at[idst_v.at[g]], add=True)
        return carry

    lax.fori_loop(0, _NCHUNK, _chunk, 0)

    plsc.subcore_barrier()

    def _copy_out(j, carry):
        start = sid * _ZROWS + j * _ZCH
        pltpu.sync_copy(acc.at[pl.ds(start, _ZCH)],
                        out.at[pl.ds(cid * _RNP + start, _ZCH)])
        return carry

    lax.fori_loop(0, _NZ, _copy_out, 0)


def _idx_body(src_ref, dst_ref, et_ref, isrc_ref, idst_ref):
    et = et_ref[...]
    isrc_ref[...] = src_ref[...] * 3 + et
    idst_ref[...] = dst_ref[...] * 3 + et


def _proj_body(x_ref, w_ref, o_ref):
    o_ref[...] = jnp.dot(x_ref[...], w_ref[...],
                         preferred_element_type=jnp.float32,
                         precision=lax.Precision.HIGHEST)


def _mid_body(h0_ref, h1_ref, w_ref, o_ref):
    h = h0_ref[...] + h1_ref[...]
    h = jnp.where(h >= 0.0, h, 0.01 * h)
    o_ref[...] = jnp.dot(h, w_ref[...],
                         preferred_element_type=jnp.float32,
                         precision=lax.Precision.HIGHEST)


def _dec_body(e0_ref, e1_ref, wd1_ref, b1_ref, wd2_ref, b2_ref, x_ref, o_ref):
    emb = e0_ref[...] + e1_ref[...]
    hid = jnp.maximum(
        jnp.dot(emb, wd1_ref[...], preferred_element_type=jnp.float32,
                precision=lax.Precision.HIGHEST) + b1_ref[...], 0.0)
    xh = jnp.dot(hid, wd2_ref[...], preferred_element_type=jnp.float32,
                 precision=lax.Precision.HIGHEST) + b2_ref[...]
    d = xh - x_ref[...]
    part = jnp.sum(d * d)

    @pl.when(pl.program_id(0) == 0)
    def _init():
        o_ref[...] = jnp.zeros_like(o_ref)

    o_ref[...] += jnp.full((1, 1), 1.0, jnp.float32) * part


def _rows(bm, cols):
    return pl.BlockSpec((bm, cols), lambda i: (i, 0))


def _full(r, c):
    return pl.BlockSpec((r, c), lambda i: (0, 0))


def kernel(x, W1, W2, Wd1, bd1, Wd2, bd2, edge_index, edge_type):
    f32 = jnp.float32
    W1p = jnp.pad(W1, ((0, 0), (0, 0), (0, _DW - _PER)))
    W1cat = jnp.transpose(W1p, (1, 0, 2)).reshape(_NFEAT, _R * _DW)
    W2p = jnp.pad(W2, ((0, 0), (0, _DW - _PER), (0, _DW - _PER)))
    W2bd = jax.scipy.linalg.block_diag(W2p[0], W2p[1], W2p[2])
    Wd1p = jnp.pad(Wd1.reshape(_R, _PER, _NEMB),
                   ((0, 0), (0, _DW - _PER), (0, 0))).reshape(_R * _DW, _NEMB)
    Wd1p = jnp.pad(Wd1p, ((0, 0), (0, _NFEAT - _NEMB)))
    b1p = jnp.pad(bd1, (0, _NFEAT - _NEMB)).reshape(1, _NFEAT)
    Wd2p = jnp.pad(Wd2, ((0, _NFEAT - _NEMB), (0, 0)))
    b2p = bd2.reshape(1, _NFEAT)

    rows2d = _E // 128
    src2 = edge_index[0].reshape(rows2d, 128)
    dst2 = edge_index[1].reshape(rows2d, 128)
    et2 = edge_type.reshape(rows2d, 128)

    isrc2, idst2 = pl.pallas_call(
        _idx_body,
        grid=(1,),
        in_specs=[_full(rows2d, 128)] * 3,
        out_specs=[_full(rows2d, 128)] * 2,
        out_shape=[jax.ShapeDtypeStruct((rows2d, 128), jnp.int32)] * 2,
    )(src2, dst2, et2)
    isrc = isrc2.reshape(_NW, _NCHUNK, _CH)
    idst = idst2.reshape(_NW, _NCHUNK, _CH)

    xp = jnp.pad(x, ((0, _NP - _N), (0, 0)))
    bm = 1280
    P = pl.pallas_call(
        _proj_body,
        grid=(_NP // bm,),
        in_specs=[_rows(bm, _NFEAT), _full(_NFEAT, _R * _DW)],
        out_specs=_rows(bm, _R * _DW),
        out_shape=jax.ShapeDtypeStruct((_NP, _R * _DW), f32),
    )(xp, W1cat)

    Hflat = _edge_pass(P.reshape(_RNP, _DW), isrc, idst)
    H0 = Hflat[:_RNP].reshape(_NP, _R * _DW)
    H1 = Hflat[_RNP:].reshape(_NP, _R * _DW)

    Q = pl.pallas_call(
        _mid_body,
        grid=(_NP // bm,),
        in_specs=[_rows(bm, _R * _DW), _rows(bm, _R * _DW),
                  _full(_R * _DW, _R * _DW)],
        out_specs=_rows(bm, _R * _DW),
        out_shape=jax.ShapeDtypeStruct((_NP, _R * _DW), f32),
    )(H0, H1, W2bd)

    Eflat = _edge_pass(Q.reshape(_RNP, _DW), isrc, idst)
    E0 = Eflat[:_RNP].reshape(_NP, _R * _DW)
    E1 = Eflat[_RNP:].reshape(_NP, _R * _DW)

    bd = 2000
    ssq = pl.pallas_call(
        _dec_body,
        grid=(_N // bd,),
        in_specs=[_rows(bd, _R * _DW), _rows(bd, _R * _DW),
                  _full(_R * _DW, _NFEAT), _full(1, _NFEAT),
                  _full(_NFEAT, _NFEAT), _full(1, _NFEAT),
                  _rows(bd, _NFEAT)],
        out_specs=_full(1, 1),
        out_shape=jax.ShapeDtypeStruct((1, 1), f32),
    )(E0, E1, Wd1p, b1p, Wd2p, b2p, x)

    sem_loss = ssq[0, 0] / (_N * _NFEAT)
    zero = jnp.asarray(0.0, dtype=f32)
    return jnp.stack([sem_loss, zero, zero, zero])


# speedup vs baseline: 15.2518x; 1.2184x over previous
import functools

import jax
import jax.numpy as jnp
from jax import lax
from jax.experimental import pallas as pl
from jax.experimental.pallas import tpu as pltpu
from jax.experimental.pallas import tpu_sc as plsc

_N = 10000
_E = 320000
_NFEAT = 128
_NEMB = 126
_PER = 42
_R = 3
_DW = 48
_RN = _R * _N
_NP = 10240
_RNP = 30720
_NW = 32
_EW = _E // _NW
_CH = 80
_NCHUNK = _EW // _CH
_ZROWS = _RNP // 16
_ZCH = 120
_NZ = _ZROWS // _ZCH
_NBUF = 4
_AHEAD = 2
_ZCOPY = _NBUF * _CH

_mesh = plsc.VectorSubcoreMesh(core_axis_name="c", subcore_axis_name="s")


@functools.partial(
    pl.kernel,
    mesh=_mesh,
    out_type=jax.ShapeDtypeStruct((2 * _RNP, _DW), jnp.float32),
    scratch_types=[
        pltpu.VMEM((_NCHUNK, _CH), jnp.int32),
        pltpu.VMEM((_NCHUNK, _CH), jnp.int32),
        pltpu.VMEM((_NBUF * _CH, _DW), jnp.float32),
        pltpu.VMEM_SHARED((_RNP, _DW), jnp.float32),
        pltpu.SemaphoreType.DMA((_NBUF,)),
        pltpu.SemaphoreType.DMA((_NBUF,)),
    ],
    compiler_params=pltpu.CompilerParams(use_tc_tiling_on_sc=False),
)
def _edge_pass(table, isrc, idst, out, isrc_v, idst_v, rows_v, acc,
               gsem, ssem):
    cid = lax.axis_index("c")
    sid = lax.axis_index("s")
    wid = cid * 16 + sid
    zero16 = jnp.zeros((16,), jnp.float32)

    pltpu.sync_copy(isrc.at[wid], isrc_v)
    pltpu.sync_copy(idst.at[wid], idst_v)

    def _buf(b):
        return rows_v.at[pl.ds(b * _CH, _CH)]

    def _gather(g, b):
        return pltpu.async_copy(table.at[isrc_v.at[g]], _buf(b), gsem.at[b])

    def _scatter_desc(g, b):
        return pltpu.make_async_copy(_buf(b), acc.at[idst_v.at[g]],
                                     ssem.at[b])

    def _zrow(r, carry):
        for cpart in range(_DW // 16):
            rows_v[r, pl.ds(cpart * 16, 16)] = zero16
        return carry

    lax.fori_loop(0, _ZCOPY, _zrow, 0)

    def _zacc(j, carry):
        pltpu.sync_copy(rows_v,
                        acc.at[pl.ds(sid * _ZROWS + j * _ZCOPY, _ZCOPY)])
        return carry

    lax.fori_loop(0, _ZROWS // _ZCOPY, _zacc, 0)

    for b0 in range(_AHEAD):
        _gather(b0, b0)

    plsc.subcore_barrier()

    def _chunk(g, carry):
        b = lax.rem(g, _NBUF)
        pltpu.make_async_copy(table.at[isrc_v.at[g]], _buf(b),
                              gsem.at[b]).wait()
        pltpu.async_copy(_buf(b), acc.at[idst_v.at[g]], ssem.at[b],
                         add=True)

        bb = lax.rem(g + _AHEAD, _NBUF)

        @pl.when(g >= _NBUF - _AHEAD)
        def _drain():
            _scatter_desc(g - (_NBUF - _AHEAD), bb).wait()

        @pl.when(g + _AHEAD < _NCHUNK)
        def _next():
            _gather(g + _AHEAD, bb)

        return carry

    lax.fori_loop(0, _NCHUNK, _chunk, 0)

    def _drain_tail(g, carry):
        _scatter_desc(g, lax.rem(g, _NBUF)).wait()
        return carry

    lax.fori_loop(_NCHUNK - (_NBUF - _AHEAD), _NCHUNK, _drain_tail, 0)

    plsc.subcore_barrier()

    def _copy_out(j, carry):
        start = sid * _ZROWS + j * _ZCH
        pltpu.sync_copy(acc.at[pl.ds(start, _ZCH)],
                        out.at[pl.ds(cid * _RNP + start, _ZCH)])
        return carry

    lax.fori_loop(0, _NZ, _copy_out, 0)


def _idx_body(src_ref, dst_ref, et_ref, isrc_ref, idst_ref):
    et = et_ref[...]
    isrc_ref[...] = src_ref[...] * 3 + et
    idst_ref[...] = dst_ref[...] * 3 + et


def _proj_body(x_ref, w_ref, o_ref):
    o_ref[...] = jnp.dot(x_ref[...], w_ref[...],
                         preferred_element_type=jnp.float32,
                         precision=lax.Precision.HIGHEST)


def _mid_body(h0_ref, h1_ref, w_ref, o_ref):
    h = h0_ref[...] + h1_ref[...]
    h = jnp.where(h >= 0.0, h, 0.01 * h)
    o_ref[...] = jnp.dot(h, w_ref[...],
                         preferred_element_type=jnp.float32,
                         precision=lax.Precision.HIGHEST)


def _dec_body(e0_ref, e1_ref, wd1_ref, b1_ref, wd2_ref, b2_ref, x_ref, o_ref):
    emb = e0_ref[...] + e1_ref[...]
    hid = jnp.maximum(
        jnp.dot(emb, wd1_ref[...], preferred_element_type=jnp.float32,
                precision=lax.Precision.HIGHEST) + b1_ref[...], 0.0)
    xh = jnp.dot(hid, wd2_ref[...], preferred_element_type=jnp.float32,
                 precision=lax.Precision.HIGHEST) + b2_ref[...]
    d = xh - x_ref[...]
    part = jnp.sum(d * d)

    @pl.when(pl.program_id(0) == 0)
    def _init():
        o_ref[...] = jnp.zeros_like(o_ref)

    o_ref[...] += jnp.full((1, 1), 1.0, jnp.float32) * part


def _rows(bm, cols):
    return pl.BlockSpec((bm, cols), lambda i: (i, 0))


def _full(r, c):
    return pl.BlockSpec((r, c), lambda i: (0, 0))


def kernel(x, W1, W2, Wd1, bd1, Wd2, bd2, edge_index, edge_type):
    f32 = jnp.float32
    W1p = jnp.pad(W1, ((0, 0), (0, 0), (0, _DW - _PER)))
    W1cat = jnp.transpose(W1p, (1, 0, 2)).reshape(_NFEAT, _R * _DW)
    W2p = jnp.pad(W2, ((0, 0), (0, _DW - _PER), (0, _DW - _PER)))
    W2bd = jax.scipy.linalg.block_diag(W2p[0], W2p[1], W2p[2])
    Wd1p = jnp.pad(Wd1.reshape(_R, _PER, _NEMB),
                   ((0, 0), (0, _DW - _PER), (0, 0))).reshape(_R * _DW, _NEMB)
    Wd1p = jnp.pad(Wd1p, ((0, 0), (0, _NFEAT - _NEMB)))
    b1p = jnp.pad(bd1, (0, _NFEAT - _NEMB)).reshape(1, _NFEAT)
    Wd2p = jnp.pad(Wd2, ((0, _NFEAT - _NEMB), (0, 0)))
    b2p = bd2.reshape(1, _NFEAT)

    rows2d = _E // 128
    src2 = edge_index[0].reshape(rows2d, 128)
    dst2 = edge_index[1].reshape(rows2d, 128)
    et2 = edge_type.reshape(rows2d, 128)

    isrc2, idst2 = pl.pallas_call(
        _idx_body,
        grid=(1,),
        in_specs=[_full(rows2d, 128)] * 3,
        out_specs=[_full(rows2d, 128)] * 2,
        out_shape=[jax.ShapeDtypeStruct((rows2d, 128), jnp.int32)] * 2,
    )(src2, dst2, et2)
    isrc = isrc2.reshape(_NW, _NCHUNK, _CH)
    idst = idst2.reshape(_NW, _NCHUNK, _CH)

    xp = jnp.pad(x, ((0, _NP - _N), (0, 0)))
    bm = 1280
    P = pl.pallas_call(
        _proj_body,
        grid=(_NP // bm,),
        in_specs=[_rows(bm, _NFEAT), _full(_NFEAT, _R * _DW)],
        out_specs=_rows(bm, _R * _DW),
        out_shape=jax.ShapeDtypeStruct((_NP, _R * _DW), f32),
    )(xp, W1cat)

    Hflat = _edge_pass(P.reshape(_RNP, _DW), isrc, idst)
    H0 = Hflat[:_RNP].reshape(_NP, _R * _DW)
    H1 = Hflat[_RNP:].reshape(_NP, _R * _DW)

    Q = pl.pallas_call(
        _mid_body,
        grid=(_NP // bm,),
        in_specs=[_rows(bm, _R * _DW), _rows(bm, _R * _DW),
                  _full(_R * _DW, _R * _DW)],
        out_specs=_rows(bm, _R * _DW),
        out_shape=jax.ShapeDtypeStruct((_NP, _R * _DW), f32),
    )(H0, H1, W2bd)

    Eflat = _edge_pass(Q.reshape(_RNP, _DW), isrc, idst)
    E0 = Eflat[:_RNP].reshape(_NP, _R * _DW)
    E1 = Eflat[_RNP:].reshape(_NP, _R * _DW)

    bd = 2000
    ssq = pl.pallas_call(
        _dec_body,
        grid=(_N // bd,),
        in_specs=[_rows(bd, _R * _DW), _rows(bd, _R * _DW),
                  _full(_R * _DW, _NFEAT), _full(1, _NFEAT),
                  _full(_NFEAT, _NFEAT), _full(1, _NFEAT),
                  _rows(bd, _NFEAT)],
        out_specs=_full(1, 1),
        out_shape=jax.ShapeDtypeStruct((1, 1), f32),
    )(E0, E1, Wd1p, b1p, Wd2p, b2p, x)

    sem_loss = ssq[0, 0] / (_N * _NFEAT)
    zero = jnp.asarray(0.0, dtype=f32)
    return jnp.stack([sem_loss, zero, zero, zero])


# speedup vs baseline: 15.4320x; 1.0118x over previous
import functools

import jax
import jax.numpy as jnp
from jax import lax
from jax.experimental import pallas as pl
from jax.experimental.pallas import tpu as pltpu
from jax.experimental.pallas import tpu_sc as plsc

_N = 10000
_E = 320000
_NFEAT = 128
_NEMB = 126
_PER = 42
_R = 3
_DW = 48
_RN = _R * _N
_NP = 10240
_RNP = 30720
_NW = 32
_EW = _E // _NW
_CH = 80
_NCHUNK = _EW // _CH
_ZROWS = _RNP // 16
_ZCH = 120
_NZ = _ZROWS // _ZCH
_NBUF = 4
_AHEAD = 2
_ZCOPY = _NBUF * _CH

_mesh = plsc.VectorSubcoreMesh(core_axis_name="c", subcore_axis_name="s")


@functools.partial(
    pl.kernel,
    mesh=_mesh,
    out_type=[jax.ShapeDtypeStruct((_RNP, _DW), jnp.float32),
              jax.ShapeDtypeStruct((_RNP, _DW), jnp.float32)],
    scratch_types=[
        pltpu.VMEM((_NCHUNK, _CH), jnp.int32),
        pltpu.VMEM((_NCHUNK, _CH), jnp.int32),
        pltpu.VMEM((_NBUF * _CH, _DW), jnp.float32),
        pltpu.VMEM_SHARED((_RNP, _DW), jnp.float32),
        pltpu.SemaphoreType.DMA((_NBUF,)),
        pltpu.SemaphoreType.DMA((_NBUF,)),
    ],
    compiler_params=pltpu.CompilerParams(use_tc_tiling_on_sc=False),
)
def _edge_pass(table, isrc, idst, out0, out1, isrc_v, idst_v, rows_v, acc,
               gsem, ssem):
    cid = lax.axis_index("c")
    sid = lax.axis_index("s")
    wid = cid * 16 + sid
    zero16 = jnp.zeros((16,), jnp.float32)

    pltpu.sync_copy(isrc.at[wid], isrc_v)
    pltpu.sync_copy(idst.at[wid], idst_v)

    def _buf(b):
        return rows_v.at[pl.ds(b * _CH, _CH)]

    def _gather(g, b):
        return pltpu.async_copy(table.at[isrc_v.at[g]], _buf(b), gsem.at[b])

    def _scatter_desc(g, b):
        return pltpu.make_async_copy(_buf(b), acc.at[idst_v.at[g]],
                                     ssem.at[b])

    def _zrow(r, carry):
        for cpart in range(_DW // 16):
            rows_v[r, pl.ds(cpart * 16, 16)] = zero16
        return carry

    lax.fori_loop(0, _ZCOPY, _zrow, 0)

    def _zacc(j, carry):
        pltpu.sync_copy(rows_v,
                        acc.at[pl.ds(sid * _ZROWS + j * _ZCOPY, _ZCOPY)])
        return carry

    lax.fori_loop(0, _ZROWS // _ZCOPY, _zacc, 0)

    for b0 in range(_AHEAD):
        _gather(b0, b0)

    plsc.subcore_barrier()

    def _chunk(g, carry):
        b = lax.rem(g, _NBUF)
        pltpu.make_async_copy(table.at[isrc_v.at[g]], _buf(b),
                              gsem.at[b]).wait()
        pltpu.async_copy(_buf(b), acc.at[idst_v.at[g]], ssem.at[b],
                         add=True)

        bb = lax.rem(g + _AHEAD, _NBUF)

        @pl.when(g >= _NBUF - _AHEAD)
        def _drain():
            _scatter_desc(g - (_NBUF - _AHEAD), bb).wait()

        @pl.when(g + _AHEAD < _NCHUNK)
        def _next():
            _gather(g + _AHEAD, bb)

        return carry

    lax.fori_loop(0, _NCHUNK, _chunk, 0)

    def _drain_tail(g, carry):
        _scatter_desc(g, lax.rem(g, _NBUF)).wait()
        return carry

    lax.fori_loop(_NCHUNK - (_NBUF - _AHEAD), _NCHUNK, _drain_tail, 0)

    plsc.subcore_barrier()

    @pl.when(cid == 0)
    def _co0():
        def _copy_out(j, carry):
            start = sid * _ZROWS + j * _ZCH
            pltpu.sync_copy(acc.at[pl.ds(start, _ZCH)],
                            out0.at[pl.ds(start, _ZCH)])
            return carry

        lax.fori_loop(0, _NZ, _copy_out, 0)

    @pl.when(cid == 1)
    def _co1():
        def _copy_out(j, carry):
            start = sid * _ZROWS + j * _ZCH
            pltpu.sync_copy(acc.at[pl.ds(start, _ZCH)],
                            out1.at[pl.ds(start, _ZCH)])
            return carry

        lax.fori_loop(0, _NZ, _copy_out, 0)


def _idx_body(src_ref, dst_ref, et_ref, isrc_ref, idst_ref):
    et = et_ref[...]
    isrc_ref[...] = src_ref[...] * 3 + et
    idst_ref[...] = dst_ref[...] * 3 + et


def _proj_body(x_ref, w_ref, o_ref):
    o_ref[...] = jnp.dot(x_ref[...], w_ref[...],
                         preferred_element_type=jnp.float32,
                         precision=lax.Precision.HIGHEST)


def _mid_body(h0_ref, h1_ref, w_ref, o_ref):
    h = h0_ref[...] + h1_ref[...]
    h = jnp.where(h >= 0.0, h, 0.01 * h)
    t = jnp.dot(h, w_ref[...], preferred_element_type=jnp.float32,
                precision=lax.Precision.HIGHEST)
    r = lax.broadcasted_iota(jnp.int32, (h.shape[0], 1), 0) % 3
    o_ref[...] = (jnp.where(r == 0, t[:, 0:_DW], 0.0)
                  + jnp.where(r == 1, t[:, _DW:2 * _DW], 0.0)
                  + jnp.where(r == 2, t[:, 2 * _DW:3 * _DW], 0.0))


def _dec_body(e0_ref, e1_ref, wd1_ref, b1_ref, wd2_ref, b2_ref, x_ref, o_ref):
    emb = e0_ref[...] + e1_ref[...]
    t = jnp.dot(emb, wd1_ref[...], preferred_element_type=jnp.float32,
                precision=lax.Precision.HIGHEST)
    r = lax.broadcasted_iota(jnp.int32, (t.shape[0], 1), 0) % 3
    u = (jnp.where(r == 0, t[:, 0:_NFEAT], 0.0)
         + jnp.where(r == 1, t[:, _NFEAT:2 * _NFEAT], 0.0)
         + jnp.where(r == 2, t[:, 2 * _NFEAT:3 * _NFEAT], 0.0))
    usum = u.reshape(t.shape[0] // 3, 3, _NFEAT).sum(axis=1)
    hid = jnp.maximum(usum + b1_ref[...], 0.0)
    xh = jnp.dot(hid, wd2_ref[...], preferred_element_type=jnp.float32,
                 precision=lax.Precision.HIGHEST) + b2_ref[...]
    d = xh - x_ref[...]
    part = jnp.sum(d * d)

    @pl.when(pl.program_id(0) == 0)
    def _init():
        o_ref[...] = jnp.zeros_like(o_ref)

    o_ref[...] += jnp.full((1, 1), 1.0, jnp.float32) * part


def _rows(bm, cols):
    return pl.BlockSpec((bm, cols), lambda i: (i, 0))


def _full(r, c):
    return pl.BlockSpec((r, c), lambda i: (0, 0))


def kernel(x, W1, W2, Wd1, bd1, Wd2, bd2, edge_index, edge_type):
    f32 = jnp.float32
    W1p = jnp.pad(W1, ((0, 0), (0, 0), (0, _DW - _PER)))
    W1cat = jnp.transpose(W1p, (1, 0, 2)).reshape(_NFEAT, _R * _DW)
    W2p = jnp.pad(W2, ((0, 0), (0, _DW - _PER), (0, _DW - _PER)))
    W2cat = jnp.concatenate([W2p[0], W2p[1], W2p[2]], axis=1)
    Wd1p = jnp.pad(Wd1.reshape(_R, _PER, _NEMB),
                   ((0, 0), (0, _DW - _PER), (0, 0)))
    Wd1p = jnp.pad(Wd1p, ((0, 0), (0, 0), (0, _NFEAT - _NEMB)))
    Wd1cat = jnp.concatenate([Wd1p[0], Wd1p[1], Wd1p[2]], axis=1)
    b1p = jnp.pad(bd1, (0, _NFEAT - _NEMB)).reshape(1, _NFEAT)
    Wd2p = jnp.pad(Wd2, ((0, _NFEAT - _NEMB), (0, 0)))
    b2p = bd2.reshape(1, _NFEAT)

    rows2d = _E // _CH
    src2 = edge_index[0].reshape(rows2d, _CH)
    dst2 = edge_index[1].reshape(rows2d, _CH)
    et2 = edge_type.reshape(rows2d, _CH)

    isrc2, idst2 = pl.pallas_call(
        _idx_body,
        grid=(1,),
        in_specs=[_full(rows2d, _CH)] * 3,
        out_specs=[_full(rows2d, _CH)] * 2,
        out_shape=[jax.ShapeDtypeStruct((rows2d, _CH), jnp.int32)] * 2,
    )(src2, dst2, et2)
    isrc = isrc2.reshape(_NW, _NCHUNK, _CH)
    idst = idst2.reshape(_NW, _NCHUNK, _CH)

    xp = jnp.pad(x, ((0, _NP - _N), (0, 0)))
    bm = 1280
    P = pl.pallas_call(
        _proj_body,
        grid=(_NP // bm,),
        in_specs=[_rows(bm, _NFEAT), _full(_NFEAT, _R * _DW)],
        out_specs=_rows(bm, _R * _DW),
        out_shape=jax.ShapeDtypeStruct((_NP, _R * _DW), f32),
    )(xp, W1cat)

    H0, H1 = _edge_pass(P.reshape(_RNP, _DW), isrc, idst)

    bq = 1920
    Q = pl.pallas_call(
        _mid_body,
        grid=(_RNP // bq,),
        in_specs=[_rows(bq, _DW), _rows(bq, _DW), _full(_DW, _R * _DW)],
        out_specs=_rows(bq, _DW),
        out_shape=jax.ShapeDtypeStruct((_RNP, _DW), f32),
    )(H0, H1, W2cat)

    E0, E1 = _edge_pass(Q, isrc, idst)

    bd = 2000
    ssq = pl.pallas_call(
        _dec_body,
        grid=(_N // bd,),
        in_specs=[_rows(3 * bd, _DW), _rows(3 * bd, _DW),
                  _full(_DW, _R * _NFEAT), _full(1, _NFEAT),
                  _full(_NFEAT, _NFEAT), _full(1, _NFEAT),
                  _rows(bd, _NFEAT)],
        out_specs=_full(1, 1),
        out_shape=jax.ShapeDtypeStruct((1, 1), f32),
    )(E0, E1, Wd1cat, b1p, Wd2p, b2p, x)

    sem_loss = ssq[0, 0] / (_N * _NFEAT)
    zero = jnp.asarray(0.0, dtype=f32)
    return jnp.stack([sem_loss, zero, zero, zero])


# speedup vs baseline: 20.8443x; 1.3507x over previous
import functools

import jax
import jax.numpy as jnp
from jax import lax
from jax.experimental import pallas as pl
from jax.experimental.pallas import tpu as pltpu
from jax.experimental.pallas import tpu_sc as plsc

_N = 10000
_E = 320000
_NFEAT = 128
_NEMB = 126
_PER = 42
_R = 3
_DW = 48
_RN = _R * _N
_NP = 10240
_RNP = 30720
_NW = 32
_EW = _E // _NW
_CH = 80
_NCHUNK = _EW // _CH
_ZROWS = _RNP // 16
_ZCH = 120
_NZ = _ZROWS // _ZCH
_NBUF = 4
_AHEAD = 2
_ZCOPY = _NBUF * _CH

_mesh = plsc.VectorSubcoreMesh(core_axis_name="c", subcore_axis_name="s")


@functools.partial(
    pl.kernel,
    mesh=_mesh,
    out_type=[jax.ShapeDtypeStruct((_RNP, _DW), jnp.float32),
              jax.ShapeDtypeStruct((_RNP, _DW), jnp.float32)],
    scratch_types=[
        pltpu.VMEM((_NCHUNK, _CH), jnp.int32),
        pltpu.VMEM((_NCHUNK, _CH), jnp.int32),
        pltpu.VMEM((_NBUF * _CH, _DW), jnp.float32),
        pltpu.VMEM_SHARED((_RNP, _DW), jnp.float32),
        pltpu.SemaphoreType.DMA((_NBUF,)),
        pltpu.SemaphoreType.DMA((_NBUF,)),
    ],
    compiler_params=pltpu.CompilerParams(use_tc_tiling_on_sc=False),
)
def _edge_pass(table, isrc, idst, out0, out1, isrc_v, idst_v, rows_v, acc,
               gsem, ssem):
    cid = lax.axis_index("c")
    sid = lax.axis_index("s")
    wid = cid * 16 + sid
    zero16 = jnp.zeros((16,), jnp.float32)

    pltpu.sync_copy(isrc.at[wid], isrc_v)
    pltpu.sync_copy(idst.at[wid], idst_v)

    def _buf(b):
        return rows_v.at[pl.ds(b * _CH, _CH)]

    def _gather(g, b):
        return pltpu.async_copy(table.at[isrc_v.at[g]], _buf(b), gsem.at[b])

    def _scatter_desc(g, b):
        return pltpu.make_async_copy(_buf(b), acc.at[idst_v.at[g]],
                                     ssem.at[b])

    def _zrow(r, carry):
        for cpart in range(_DW // 16):
            rows_v[r, pl.ds(cpart * 16, 16)] = zero16
        return carry

    lax.fori_loop(0, _ZCOPY, _zrow, 0)

    def _zacc(j, carry):
        pltpu.sync_copy(rows_v,
                        acc.at[pl.ds(sid * _ZROWS + j * _ZCOPY, _ZCOPY)])
        return carry

    lax.fori_loop(0, _ZROWS // _ZCOPY, _zacc, 0)

    for b0 in range(_AHEAD):
        _gather(b0, b0)

    plsc.subcore_barrier()

    def _chunk(g, carry):
        b = lax.rem(g, _NBUF)
        pltpu.make_async_copy(table.at[isrc_v.at[g]], _buf(b),
                              gsem.at[b]).wait()
        pltpu.async_copy(_buf(b), acc.at[idst_v.at[g]], ssem.at[b],
                         add=True)

        bb = lax.rem(g + _AHEAD, _NBUF)

        @pl.when(g >= _NBUF - _AHEAD)
        def _drain():
            _scatter_desc(g - (_NBUF - _AHEAD), bb).wait()

        @pl.when(g + _AHEAD < _NCHUNK)
        def _next():
            _gather(g + _AHEAD, bb)

        return carry

    lax.fori_loop(0, _NCHUNK, _chunk, 0)

    def _drain_tail(g, carry):
        _scatter_desc(g, lax.rem(g, _NBUF)).wait()
        return carry

    lax.fori_loop(_NCHUNK - (_NBUF - _AHEAD), _NCHUNK, _drain_tail, 0)

    plsc.subcore_barrier()

    @pl.when(cid == 0)
    def _co0():
        def _copy_out(j, carry):
            start = sid * _ZROWS + j * _ZCH
            pltpu.sync_copy(acc.at[pl.ds(start, _ZCH)],
                            out0.at[pl.ds(start, _ZCH)])
            return carry

        lax.fori_loop(0, _NZ, _copy_out, 0)

    @pl.when(cid == 1)
    def _co1():
        def _copy_out(j, carry):
            start = sid * _ZROWS + j * _ZCH
            pltpu.sync_copy(acc.at[pl.ds(start, _ZCH)],
                            out1.at[pl.ds(start, _ZCH)])
            return carry

        lax.fori_loop(0, _NZ, _copy_out, 0)


def _idx_body(src_ref, dst_ref, et_ref, isrc_ref, idst_ref):
    et = et_ref[...]
    isrc_ref[...] = src_ref[...] * 3 + et
    idst_ref[...] = dst_ref[...] * 3 + et


def _proj_body(x_ref, w_ref, o_ref):
    o_ref[...] = jnp.dot(x_ref[...], w_ref[...],
                         preferred_element_type=jnp.float32)


def _mid_body(h0_ref, h1_ref, w_ref, o_ref):
    h = h0_ref[...] + h1_ref[...]
    h = jnp.where(h >= 0.0, h, 0.01 * h)
    o_ref[...] = jnp.dot(h, w_ref[...], preferred_element_type=jnp.float32)


def _dec_body(e0_ref, e1_ref, wd1_ref, b1_ref, wd2_ref, b2_ref, x_ref, o_ref):
    emb = e0_ref[...] + e1_ref[...]
    hid = jnp.maximum(
        jnp.dot(emb, wd1_ref[...], preferred_element_type=jnp.float32)
        + b1_ref[...], 0.0)
    xh = jnp.dot(hid, wd2_ref[...],
                 preferred_element_type=jnp.float32) + b2_ref[...]
    d = xh - x_ref[...]
    part = jnp.sum(d * d)

    @pl.when(pl.program_id(0) == 0)
    def _init():
        o_ref[...] = jnp.zeros_like(o_ref)

    o_ref[...] += jnp.full((1, 1), 1.0, jnp.float32) * part


def _rows(bm, cols):
    return pl.BlockSpec((bm, cols), lambda i: (i, 0))


def _full(r, c):
    return pl.BlockSpec((r, c), lambda i: (0, 0))


def kernel(x, W1, W2, Wd1, bd1, Wd2, bd2, edge_index, edge_type):
    f32 = jnp.float32
    W1p = jnp.pad(W1, ((0, 0), (0, 0), (0, _DW - _PER)))
    W1cat = jnp.transpose(W1p, (1, 0, 2)).reshape(_NFEAT, _R * _DW)
    W2p = jnp.pad(W2, ((0, 0), (0, _DW - _PER), (0, _DW - _PER)))
    W2bd = jax.scipy.linalg.block_diag(W2p[0], W2p[1], W2p[2])
    Wd1p = jnp.pad(Wd1.reshape(_R, _PER, _NEMB),
                   ((0, 0), (0, _DW - _PER), (0, 0))).reshape(_R * _DW, _NEMB)
    Wd1p = jnp.pad(Wd1p, ((0, 0), (0, _NFEAT - _NEMB)))
    b1p = jnp.pad(bd1, (0, _NFEAT - _NEMB)).reshape(1, _NFEAT)
    Wd2p = jnp.pad(Wd2, ((0, _NFEAT - _NEMB), (0, 0)))
    b2p = bd2.reshape(1, _NFEAT)

    rows2d = _E // _CH
    src2 = edge_index[0].reshape(rows2d, _CH)
    dst2 = edge_index[1].reshape(rows2d, _CH)
    et2 = edge_type.reshape(rows2d, _CH)

    isrc2, idst2 = pl.pallas_call(
        _idx_body,
        grid=(1,),
        in_specs=[_full(rows2d, _CH)] * 3,
        out_specs=[_full(rows2d, _CH)] * 2,
        out_shape=[jax.ShapeDtypeStruct((rows2d, _CH), jnp.int32)] * 2,
    )(src2, dst2, et2)
    isrc = isrc2.reshape(_NW, _NCHUNK, _CH)
    idst = idst2.reshape(_NW, _NCHUNK, _CH)

    xp = jnp.pad(x, ((0, _NP - _N), (0, 0)))
    bm = 1280
    P = pl.pallas_call(
        _proj_body,
        grid=(_NP // bm,),
        in_specs=[_rows(bm, _NFEAT), _full(_NFEAT, _R * _DW)],
        out_specs=_rows(bm, _R * _DW),
        out_shape=jax.ShapeDtypeStruct((_NP, _R * _DW), f32),
    )(xp, W1cat)

    H0f, H1f = _edge_pass(P.reshape(_RNP, _DW), isrc, idst)
    H0 = H0f.reshape(_NP, _R * _DW)
    H1 = H1f.reshape(_NP, _R * _DW)

    Q = pl.pallas_call(
        _mid_body,
        grid=(_NP // bm,),
        in_specs=[_rows(bm, _R * _DW), _rows(bm, _R * _DW),
                  _full(_R * _DW, _R * _DW)],
        out_specs=_rows(bm, _R * _DW),
        out_shape=jax.ShapeDtypeStruct((_NP, _R * _DW), f32),
    )(H0, H1, W2bd)

    E0f, E1f = _edge_pass(Q.reshape(_RNP, _DW), isrc, idst)
    E0 = E0f.reshape(_NP, _R * _DW)
    E1 = E1f.reshape(_NP, _R * _DW)

    bd = 2000
    ssq = pl.pallas_call(
        _dec_body,
        grid=(_N // bd,),
        in_specs=[_rows(bd, _R * _DW), _rows(bd, _R * _DW),
                  _full(_R * _DW, _NFEAT), _full(1, _NFEAT),
                  _full(_NFEAT, _NFEAT), _full(1, _NFEAT),
                  _rows(bd, _NFEAT)],
        out_specs=_full(1, 1),
        out_shape=jax.ShapeDtypeStruct((1, 1), f32),
    )(E0, E1, Wd1p, b1p, Wd2p, b2p, x)

    sem_loss = ssq[0, 0] / (_N * _NFEAT)
    zero = jnp.asarray(0.0, dtype=f32)
    return jnp.stack([sem_loss, zero, zero, zero])


# speedup vs baseline: 23.8896x; 1.1461x over previous
import functools

import jax
import jax.numpy as jnp
from jax import lax
from jax.experimental import pallas as pl
from jax.experimental.pallas import tpu as pltpu
from jax.experimental.pallas import tpu_sc as plsc

_N = 10000
_E = 320000
_NFEAT = 128
_NEMB = 126
_PER = 42
_R = 3
_DW = 48
_RN = _R * _N
_NP = 10240
_RNP = 30720
_NW = 32
_EW = _E // _NW
_CH = 80
_NCHUNK = _EW // _CH
_ZROWS = _RNP // 16
_ZCH = 120
_NZ = _ZROWS // _ZCH
_NBUF = 4
_AHEAD = 2
_ZCOPY = 384

_mesh = plsc.VectorSubcoreMesh(core_axis_name="c", subcore_axis_name="s")


@functools.partial(
    pl.kernel,
    mesh=_mesh,
    out_type=[jax.ShapeDtypeStruct((_RNP, _DW), jnp.float32),
              jax.ShapeDtypeStruct((_RNP, _DW), jnp.float32)],
    scratch_types=[
        pltpu.VMEM((_NCHUNK, _CH), jnp.int32),
        pltpu.VMEM((_NCHUNK, _CH), jnp.int32),
        pltpu.VMEM((_NBUF * _CH, _DW), jnp.float32),
        pltpu.VMEM_SHARED((_RNP, _DW), jnp.float32),
        pltpu.SemaphoreType.DMA((_NBUF,)),
        pltpu.SemaphoreType.DMA((_NBUF,)),
        pltpu.SemaphoreType.DMA,
    ],
    compiler_params=pltpu.CompilerParams(use_tc_tiling_on_sc=False),
)
def _edge_pass(table, isrc, idst, out0, out1, isrc_v, idst_v, rows_v, acc,
               gsem, ssem, osem):
    cid = lax.axis_index("c")
    sid = lax.axis_index("s")
    wid = cid * 16 + sid
    zero16 = jnp.zeros((16,), jnp.float32)

    stage0 = pltpu.async_copy(isrc.at[wid], isrc_v, osem)
    stage1 = pltpu.async_copy(idst.at[wid], idst_v, osem)

    def _buf(b):
        return rows_v.at[pl.ds(b * _CH, _CH)]

    def _gather(g, b):
        return pltpu.async_copy(table.at[isrc_v.at[g]], _buf(b), gsem.at[b])

    def _scatter_desc(g, b):
        return pltpu.make_async_copy(_buf(b), acc.at[idst_v.at[g]],
                                     ssem.at[b])

    def _zrow(r, carry):
        for cpart in range(_DW // 16):
            rows_v[r, pl.ds(cpart * 16, 16)] = zero16
        return carry

    lax.fori_loop(0, _ZCOPY, _zrow, 0)

    def _zacc(j, carry):
        pltpu.sync_copy(rows_v.at[pl.ds(0, _ZCOPY)],
                        acc.at[pl.ds(sid * _ZROWS + j * _ZCOPY, _ZCOPY)])
        return carry

    lax.fori_loop(0, _ZROWS // _ZCOPY, _zacc, 0)

    stage0.wait()
    stage1.wait()
    for b0 in range(_AHEAD):
        _gather(b0, b0)

    plsc.subcore_barrier()

    def _chunk(g, carry):
        b = lax.rem(g, _NBUF)
        pltpu.make_async_copy(table.at[isrc_v.at[g]], _buf(b),
                              gsem.at[b]).wait()
        pltpu.async_copy(_buf(b), acc.at[idst_v.at[g]], ssem.at[b],
                         add=True)

        bb = lax.rem(g + _AHEAD, _NBUF)

        @pl.when(g >= _NBUF - _AHEAD)
        def _drain():
            _scatter_desc(g - (_NBUF - _AHEAD), bb).wait()

        @pl.when(g + _AHEAD < _NCHUNK)
        def _next():
            _gather(g + _AHEAD, bb)

        return carry

    lax.fori_loop(0, _NCHUNK, _chunk, 0)

    def _drain_tail(g, carry):
        _scatter_desc(g, lax.rem(g, _NBUF)).wait()
        return carry

    lax.fori_loop(_NCHUNK - (_NBUF - _AHEAD), _NCHUNK, _drain_tail, 0)

    plsc.subcore_barrier()

    @pl.when(cid == 0)
    def _co0():
        def _copy_out(j, carry):
            start = sid * _ZROWS + j * _ZCH
            pltpu.async_copy(acc.at[pl.ds(start, _ZCH)],
                             out0.at[pl.ds(start, _ZCH)], osem)
            return carry

        lax.fori_loop(0, _NZ, _copy_out, 0)

        def _wait_out(j, carry):
            start = sid * _ZROWS + j * _ZCH
            pltpu.make_async_copy(acc.at[pl.ds(start, _ZCH)],
                                  out0.at[pl.ds(start, _ZCH)], osem).wait()
            return carry

        lax.fori_loop(0, _NZ, _wait_out, 0)

    @pl.when(cid == 1)
    def _co1():
        def _copy_out(j, carry):
            start = sid * _ZROWS + j * _ZCH
            pltpu.async_copy(acc.at[pl.ds(start, _ZCH)],
                             out1.at[pl.ds(start, _ZCH)], osem)
            return carry

        lax.fori_loop(0, _NZ, _copy_out, 0)

        def _wait_out(j, carry):
            start = sid * _ZROWS + j * _ZCH
            pltpu.make_async_copy(acc.at[pl.ds(start, _ZCH)],
                                  out1.at[pl.ds(start, _ZCH)], osem).wait()
            return carry

        lax.fori_loop(0, _NZ, _wait_out, 0)


def _idx_body(ei_ref, et_ref, isrc_ref, idst_ref):
    et = et_ref[...]
    rows = et.shape[0]
    src = ei_ref[0, :].reshape(rows, 128)
    dst = ei_ref[1, :].reshape(rows, 128)
    isrc_ref[...] = src * 3 + et
    idst_ref[...] = dst * 3 + et


def _proj_body(x_ref, w_ref, o_ref):
    o_ref[...] = jnp.dot(x_ref[...], w_ref[...],
                         preferred_element_type=jnp.float32)


def _mid_body(h0_ref, h1_ref, w_ref, o_ref):
    h = h0_ref[...] + h1_ref[...]
    h = jnp.where(h >= 0.0, h, 0.01 * h)
    o_ref[...] = jnp.dot(h, w_ref[...], preferred_element_type=jnp.float32)


def _dec_body(e0_ref, e1_ref, wd1_ref, b1_ref, wd2_ref, b2_ref, x_ref, o_ref):
    emb = e0_ref[...] + e1_ref[...]
    hid = jnp.maximum(
        jnp.dot(emb, wd1_ref[...], preferred_element_type=jnp.float32)
        + b1_ref[...], 0.0)
    xh = jnp.dot(hid, wd2_ref[...],
                 preferred_element_type=jnp.float32) + b2_ref[...]
    d = xh - x_ref[...]
    part = jnp.sum(d * d)

    @pl.when(pl.program_id(0) == 0)
    def _init():
        o_ref[...] = jnp.zeros_like(o_ref)

    o_ref[...] += jnp.full((1, 1), 1.0, jnp.float32) * part


def _rows(bm, cols):
    return pl.BlockSpec((bm, cols), lambda i: (i, 0))


def _full(r, c):
    return pl.BlockSpec((r, c), lambda i: (0, 0))


def kernel(x, W1, W2, Wd1, bd1, Wd2, bd2, edge_index, edge_type):
    f32 = jnp.float32
    W1p = jnp.pad(W1, ((0, 0), (0, 0), (0, _DW - _PER)))
    W1cat = jnp.transpose(W1p, (1, 0, 2)).reshape(_NFEAT, _R * _DW)
    W2p = jnp.pad(W2, ((0, 0), (0, _DW - _PER), (0, _DW - _PER)))
    W2bd = jax.scipy.linalg.block_diag(W2p[0], W2p[1], W2p[2])
    Wd1p = jnp.pad(Wd1.reshape(_R, _PER, _NEMB),
                   ((0, 0), (0, _DW - _PER), (0, 0))).reshape(_R * _DW, _NEMB)
    Wd1p = jnp.pad(Wd1p, ((0, 0), (0, _NFEAT - _NEMB)))
    b1p = jnp.pad(bd1, (0, _NFEAT - _NEMB)).reshape(1, _NFEAT)
    Wd2p = jnp.pad(Wd2, ((0, _NFEAT - _NEMB), (0, 0)))
    b2p = bd2.reshape(1, _NFEAT)

    rows2d = _E // 128
    et2 = edge_type.reshape(rows2d, 128)

    isrc2, idst2 = pl.pallas_call(
        _idx_body,
        grid=(1,),
        in_specs=[_full(2, _E), _full(rows2d, 128)],
        out_specs=[_full(rows2d, 128)] * 2,
        out_shape=[jax.ShapeDtypeStruct((rows2d, 128), jnp.int32)] * 2,
    )(edge_index, et2)
    isrc = isrc2.reshape(_NW, _NCHUNK, _CH)
    idst = idst2.reshape(_NW, _NCHUNK, _CH)

    xp = jnp.pad(x, ((0, _NP - _N), (0, 0)))
    bm = 1280
    P = pl.pallas_call(
        _proj_body,
        grid=(_NP // bm,),
        in_specs=[_rows(bm, _NFEAT), _full(_NFEAT, _R * _DW)],
        out_specs=_rows(bm, _R * _DW),
        out_shape=jax.ShapeDtypeStruct((_NP, _R * _DW), f32),
    )(xp, W1cat)

    H0f, H1f = _edge_pass(P.reshape(_RNP, _DW), isrc, idst)
    H0 = H0f.reshape(_NP, _R * _DW)
    H1 = H1f.reshape(_NP, _R * _DW)

    Q = pl.pallas_call(
        _mid_body,
        grid=(_NP // bm,),
        in_specs=[_rows(bm, _R * _DW), _rows(bm, _R * _DW),
                  _full(_R * _DW, _R * _DW)],
        out_specs=_rows(bm, _R * _DW),
        out_shape=jax.ShapeDtypeStruct((_NP, _R * _DW), f32),
    )(H0, H1, W2bd)

    E0f, E1f = _edge_pass(Q.reshape(_RNP, _DW), isrc, idst)
    E0 = E0f.reshape(_NP, _R * _DW)
    E1 = E1f.reshape(_NP, _R * _DW)

    bd = 2000
    ssq = pl.pallas_call(
        _dec_body,
        grid=(_N // bd,),
        in_specs=[_rows(bd, _R * _DW), _rows(bd, _R * _DW),
                  _full(_R * _DW, _NFEAT), _full(1, _NFEAT),
                  _full(_NFEAT, _NFEAT), _full(1, _NFEAT),
                  _rows(bd, _NFEAT)],
        out_specs=_full(1, 1),
        out_shape=jax.ShapeDtypeStruct((1, 1), f32),
    )(E0, E1, Wd1p, b1p, Wd2p, b2p, x)

    sem_loss = ssq[0, 0] / (_N * _NFEAT)
    zero = jnp.asarray(0.0, dtype=f32)
    return jnp.stack([sem_loss, zero, zero, zero])
